# Initial kernel scaffold; baseline (speedup 1.0000x reference)
import jax
import jax.numpy as jnp
from jax.experimental import pallas as pl


def kernel(h, edge_index):
    raise NotImplementedError("write your pallas kernel here")


# speedup vs baseline: 1.1008x; 1.1008x over previous
import functools

import jax
import jax.numpy as jnp
from jax import lax
from jax.experimental import pallas as pl
from jax.experimental.pallas import tpu as pltpu
from jax.experimental.pallas import tpu_sc as plsc

NC, NS, L = 2, 16, 16
NW = NC * NS
E = 320000
D = 128
EPW = E // NW
C = 80
NCHUNK = EPW // C


@functools.partial(
    pl.kernel,
    out_type=jax.ShapeDtypeStruct((E,), jnp.float32),
    mesh=plsc.VectorSubcoreMesh(
        core_axis_name="c", subcore_axis_name="s",
        num_cores=NC, num_subcores=NS),
    scratch_types=[
        pltpu.VMEM((C,), jnp.int32),
        pltpu.VMEM((C,), jnp.int32),
        pltpu.VMEM((C, D), jnp.float32),
        pltpu.VMEM((C, D), jnp.float32),
        pltpu.VMEM((C,), jnp.float32),
        pltpu.SemaphoreType.DMA,
        pltpu.SemaphoreType.DMA,
    ],
    compiler_params=pltpu.CompilerParams(needs_layout_passes=False),
)
def _edge_dot(h_hbm, src_hbm, dst_hbm, out_hbm,
              sidx_v, didx_v, srows_v, drows_v, out_v, sem_s, sem_d):
    wid = lax.axis_index("s") * NC + lax.axis_index("c")
    wbase = wid * EPW

    def chunk(i, carry):
        base = wbase + i * C
        pltpu.sync_copy(src_hbm.at[pl.ds(base, C)], sidx_v)
        pltpu.sync_copy(dst_hbm.at[pl.ds(base, C)], didx_v)
        cp_s = pltpu.make_async_copy(h_hbm.at[sidx_v], srows_v, sem_s)
        cp_d = pltpu.make_async_copy(h_hbm.at[didx_v], drows_v, sem_d)
        cp_s.start()
        cp_d.start()
        cp_s.wait()
        cp_d.wait()
        for g in range(C // L):
            ids = lax.iota(jnp.int32, L) + (g * L)
            acc = jnp.zeros((L,), jnp.float32)
            for f in range(D):
                fv = jnp.full((L,), f, jnp.int32)
                s = plsc.load_gather(srows_v, [ids, fv])
                d = plsc.load_gather(drows_v, [ids, fv])
                acc = acc + s * d
            out_v[pl.ds(g * L, L)] = acc
        pltpu.sync_copy(out_v, out_hbm.at[pl.ds(base, C)])
        return carry

    lax.fori_loop(0, NCHUNK, chunk, 0)


def kernel(h, edge_index):
    src = edge_index[0].astype(jnp.int32)
    dst = edge_index[1].astype(jnp.int32)
    return _edge_dot(h, src, dst)


# speedup vs baseline: 1.1692x; 1.0622x over previous
import functools

import jax
import jax.numpy as jnp
from jax import lax
from jax.experimental import pallas as pl
from jax.experimental.pallas import tpu as pltpu
from jax.experimental.pallas import tpu_sc as plsc

NC, NS, L = 2, 16, 16
NW = NC * NS
E = 320000
D = 128
EPW = E // NW
C = 80
NCHUNK = EPW // C
P = 2


@functools.partial(
    pl.kernel,
    out_type=jax.ShapeDtypeStruct((E,), jnp.float32),
    mesh=plsc.VectorSubcoreMesh(
        core_axis_name="c", subcore_axis_name="s",
        num_cores=NC, num_subcores=NS),
    scratch_types=[
        pltpu.VMEM((2, C), jnp.int32),
        pltpu.VMEM((2, C), jnp.int32),
        pltpu.VMEM((C, D), jnp.float32),
        pltpu.VMEM((C, D), jnp.float32),
        pltpu.VMEM((C, D), jnp.float32),
        pltpu.VMEM((C, D), jnp.float32),
        pltpu.VMEM((C,), jnp.float32),
        pltpu.VMEM((C,), jnp.float32),
        pltpu.SemaphoreType.DMA,
        pltpu.SemaphoreType.DMA,
        pltpu.SemaphoreType.DMA,
        pltpu.SemaphoreType.DMA,
        pltpu.SemaphoreType.DMA,
        pltpu.SemaphoreType.DMA,
        pltpu.SemaphoreType.DMA,
        pltpu.SemaphoreType.DMA,
    ],
    compiler_params=pltpu.CompilerParams(needs_layout_passes=False),
)
def _edge_dot(h_hbm, src_hbm, dst_hbm, out_hbm,
              idx0, idx1, s0, d0, s1, d1, o0, o1,
              si0, ss0, sd0, so0, si1, ss1, sd1, so1):
    wid = lax.axis_index("s") * NC + lax.axis_index("c")
    wbase = wid * EPW
    bufs = ((idx0, s0, d0, o0, si0, ss0, sd0, so0),
            (idx1, s1, d1, o1, si1, ss1, sd1, so1))

    def idx_copies(c, buf):
        idx, _, _, _, si = buf[:5]
        base = wbase + c * C
        return (pltpu.make_async_copy(src_hbm.at[pl.ds(base, C)],
                                      idx.at[0], si),
                pltpu.make_async_copy(dst_hbm.at[pl.ds(base, C)],
                                      idx.at[1], si))

    def gathers(buf):
        idx, srows, drows, _, _, ss, sd = buf[:7]
        return (pltpu.make_async_copy(h_hbm.at[idx.at[0]], srows, ss),
                pltpu.make_async_copy(h_hbm.at[idx.at[1]], drows, sd))

    def out_store(c, buf):
        outb, so = buf[3], buf[7]
        return pltpu.make_async_copy(
            outb, out_hbm.at[pl.ds(wbase + c * C, C)], so)

    def process(c, buf):
        idx, srows, drows, outb = buf[:4]
        cs, cd = gathers(buf)
        cs.wait()
        cd.wait()
        @pl.when(c + P < NCHUNK)
        def _():
            ia, ib = idx_copies(c + P, buf)
            ia.start()
            ib.start()
        @pl.when(c >= P)
        def _():
            out_store(c, buf).wait()

        def grp(g, carry):
            ids = lax.iota(jnp.int32, L) + g * L
            a0 = jnp.zeros((L,), jnp.float32)
            a1 = jnp.zeros((L,), jnp.float32)
            a2 = jnp.zeros((L,), jnp.float32)
            a3 = jnp.zeros((L,), jnp.float32)
            for f in range(0, D, 4):
                for j in range(4):
                    fv = jnp.full((L,), f + j, jnp.int32)
                    s = plsc.load_gather(srows, [ids, fv])
                    d = plsc.load_gather(drows, [ids, fv])
                    if j == 0:
                        a0 = a0 + s * d
                    elif j == 1:
                        a1 = a1 + s * d
                    elif j == 2:
                        a2 = a2 + s * d
                    else:
                        a3 = a3 + s * d
            outb[pl.ds(g * L, L)] = (a0 + a1) + (a2 + a3)
            return carry

        lax.fori_loop(0, C // L, grp, 0)
        out_store(c, buf).start()
        @pl.when(c + P < NCHUNK)
        def _():
            ia, ib = idx_copies(c + P, buf)
            ia.wait()
            ib.wait()
            ns, nd = gathers(buf)
            ns.start()
            nd.start()

    c0a, c0b = idx_copies(0, bufs[0])
    c1a, c1b = idx_copies(1, bufs[1])
    c0a.start()
    c0b.start()
    c1a.start()
    c1b.start()
    c0a.wait()
    c0b.wait()
    g0s, g0d = gathers(bufs[0])
    g0s.start()
    g0d.start()
    c1a.wait()
    c1b.wait()
    g1s, g1d = gathers(bufs[1])
    g1s.start()
    g1d.start()

    def pair(k, carry):
        process(2 * k, bufs[0])
        process(2 * k + 1, bufs[1])
        return carry

    lax.fori_loop(0, (NCHUNK - 1) // 2, pair, 0)
    process(NCHUNK - 1, bufs[0])
    out_store(0, bufs[0]).wait()
    out_store(0, bufs[1]).wait()


def kernel(h, edge_index):
    src = edge_index[0].astype(jnp.int32)
    dst = edge_index[1].astype(jnp.int32)
    return _edge_dot(h, src, dst)


# speedup vs baseline: 3.0049x; 2.5699x over previous
import functools

import jax
import jax.numpy as jnp
from jax import lax
from jax.experimental import pallas as pl
from jax.experimental.pallas import tpu as pltpu
from jax.experimental.pallas import tpu_sc as plsc

NC, NS, L = 2, 16, 16
NW = NC * NS
E = 320000
D = 128
EPW = E // NW
C = 80
NCHUNK = EPW // C
P = 2


@functools.partial(
    pl.kernel,
    out_type=jax.ShapeDtypeStruct((E,), jnp.float32),
    mesh=plsc.VectorSubcoreMesh(
        core_axis_name="c", subcore_axis_name="s",
        num_cores=NC, num_subcores=NS),
    scratch_types=[
        pltpu.VMEM((2, C), jnp.int32),
        pltpu.VMEM((2, C), jnp.int32),
        pltpu.VMEM((C, D), jnp.float32),
        pltpu.VMEM((C, D), jnp.float32),
        pltpu.VMEM((C, D), jnp.float32),
        pltpu.VMEM((C, D), jnp.float32),
        pltpu.VMEM((C,), jnp.float32),
        pltpu.VMEM((C,), jnp.float32),
        pltpu.SemaphoreType.DMA,
        pltpu.SemaphoreType.DMA,
        pltpu.SemaphoreType.DMA,
        pltpu.SemaphoreType.DMA,
        pltpu.SemaphoreType.DMA,
        pltpu.SemaphoreType.DMA,
        pltpu.SemaphoreType.DMA,
        pltpu.SemaphoreType.DMA,
    ],
    compiler_params=pltpu.CompilerParams(needs_layout_passes=False),
)
def _edge_dot(h_hbm, src_hbm, dst_hbm, out_hbm,
              idx0, idx1, s0, d0, s1, d1, o0, o1,
              si0, ss0, sd0, so0, si1, ss1, sd1, so1):
    wid = lax.axis_index("s") * NC + lax.axis_index("c")
    wbase = wid * EPW
    bufs = ((idx0, s0, d0, o0, si0, ss0, sd0, so0),
            (idx1, s1, d1, o1, si1, ss1, sd1, so1))

    def idx_copies(c, buf):
        idx, _, _, _, si = buf[:5]
        base = wbase + c * C
        return (pltpu.make_async_copy(src_hbm.at[pl.ds(base, C)],
                                      idx.at[0], si),
                pltpu.make_async_copy(dst_hbm.at[pl.ds(base, C)],
                                      idx.at[1], si))

    def gathers(buf):
        idx, srows, drows, _, _, ss, sd = buf[:7]
        return (pltpu.make_async_copy(h_hbm.at[idx.at[0]], srows, ss),
                pltpu.make_async_copy(h_hbm.at[idx.at[1]], drows, sd))

    def out_store(c, buf):
        outb, so = buf[3], buf[7]
        return pltpu.make_async_copy(
            outb, out_hbm.at[pl.ds(wbase + c * C, C)], so)

    def process(c, buf):
        idx, srows, drows, outb = buf[:4]
        cs, cd = gathers(buf)
        cs.wait()
        cd.wait()
        @pl.when(c + P < NCHUNK)
        def _():
            ia, ib = idx_copies(c + P, buf)
            ia.start()
            ib.start()
        @pl.when(c >= P)
        def _():
            out_store(c, buf).wait()

        def grp(g, carry):
            lane = lax.iota(jnp.int32, L)
            ids = lane + g * L
            a0 = jnp.zeros((L,), jnp.float32)
            a1 = jnp.zeros((L,), jnp.float32)
            a2 = jnp.zeros((L,), jnp.float32)
            a3 = jnp.zeros((L,), jnp.float32)
            for t in range(D):
                fv = (lane + t) & (D - 1)
                s = plsc.load_gather(srows, [ids, fv])
                d = plsc.load_gather(drows, [ids, fv])
                if t % 4 == 0:
                    a0 = a0 + s * d
                elif t % 4 == 1:
                    a1 = a1 + s * d
                elif t % 4 == 2:
                    a2 = a2 + s * d
                else:
                    a3 = a3 + s * d
            outb[pl.ds(g * L, L)] = (a0 + a1) + (a2 + a3)
            return carry

        lax.fori_loop(0, C // L, grp, 0)
        out_store(c, buf).start()
        @pl.when(c + P < NCHUNK)
        def _():
            ia, ib = idx_copies(c + P, buf)
            ia.wait()
            ib.wait()
            ns, nd = gathers(buf)
            ns.start()
            nd.start()

    c0a, c0b = idx_copies(0, bufs[0])
    c1a, c1b = idx_copies(1, bufs[1])
    c0a.start()
    c0b.start()
    c1a.start()
    c1b.start()
    c0a.wait()
    c0b.wait()
    g0s, g0d = gathers(bufs[0])
    g0s.start()
    g0d.start()
    c1a.wait()
    c1b.wait()
    g1s, g1d = gathers(bufs[1])
    g1s.start()
    g1d.start()

    def pair(k, carry):
        process(2 * k, bufs[0])
        process(2 * k + 1, bufs[1])
        return carry

    lax.fori_loop(0, (NCHUNK - 1) // 2, pair, 0)
    process(NCHUNK - 1, bufs[0])
    out_store(0, bufs[0]).wait()
    out_store(0, bufs[1]).wait()


def kernel(h, edge_index):
    src = edge_index[0].astype(jnp.int32)
    dst = edge_index[1].astype(jnp.int32)
    return _edge_dot(h, src, dst)


# speedup vs baseline: 4.3432x; 1.4454x over previous
import functools

import jax
import jax.numpy as jnp
from jax import lax
from jax.experimental import pallas as pl
from jax.experimental.pallas import tpu as pltpu
from jax.experimental.pallas import tpu_sc as plsc

NC, NS, L = 2, 16, 16
NW = NC * NS
E = 320000
D = 128
EPW = E // NW
C = 80
NCHUNK = EPW // C
P = 2
DP = D // 2


@functools.partial(
    pl.kernel,
    out_type=jax.ShapeDtypeStruct((E,), jnp.float32),
    mesh=plsc.VectorSubcoreMesh(
        core_axis_name="c", subcore_axis_name="s",
        num_cores=NC, num_subcores=NS),
    scratch_types=[
        pltpu.VMEM((2, C), jnp.int32),
        pltpu.VMEM((2, C), jnp.int32),
        pltpu.VMEM((C, DP), jnp.int32),
        pltpu.VMEM((C, DP), jnp.int32),
        pltpu.VMEM((C, DP), jnp.int32),
        pltpu.VMEM((C, DP), jnp.int32),
        pltpu.VMEM((C,), jnp.float32),
        pltpu.VMEM((C,), jnp.float32),
        pltpu.SemaphoreType.DMA,
        pltpu.SemaphoreType.DMA,
        pltpu.SemaphoreType.DMA,
        pltpu.SemaphoreType.DMA,
        pltpu.SemaphoreType.DMA,
        pltpu.SemaphoreType.DMA,
        pltpu.SemaphoreType.DMA,
        pltpu.SemaphoreType.DMA,
    ],
    compiler_params=pltpu.CompilerParams(
        needs_layout_passes=False, use_tc_tiling_on_sc=False),
)
def _edge_dot(h_hbm, src_hbm, dst_hbm, out_hbm,
              idx0, idx1, s0, d0, s1, d1, o0, o1,
              si0, ss0, sd0, so0, si1, ss1, sd1, so1):
    wid = lax.axis_index("s") * NC + lax.axis_index("c")
    wbase = wid * EPW
    bufs = ((idx0, s0, d0, o0, si0, ss0, sd0, so0),
            (idx1, s1, d1, o1, si1, ss1, sd1, so1))

    def idx_copies(c, buf):
        idx, _, _, _, si = buf[:5]
        base = wbase + c * C
        return (pltpu.make_async_copy(src_hbm.at[pl.ds(base, C)],
                                      idx.at[0], si),
                pltpu.make_async_copy(dst_hbm.at[pl.ds(base, C)],
                                      idx.at[1], si))

    def gathers(buf):
        idx, srows, drows, _, _, ss, sd = buf[:7]
        return (pltpu.make_async_copy(h_hbm.at[idx.at[0]], srows, ss),
                pltpu.make_async_copy(h_hbm.at[idx.at[1]], drows, sd))

    def out_store(c, buf):
        outb, so = buf[3], buf[7]
        return pltpu.make_async_copy(
            outb, out_hbm.at[pl.ds(wbase + c * C, C)], so)

    def process(c, buf):
        idx, srows, drows, outb = buf[:4]
        cs, cd = gathers(buf)
        cs.wait()
        cd.wait()
        @pl.when(c + P < NCHUNK)
        def _():
            ia, ib = idx_copies(c + P, buf)
            ia.start()
            ib.start()
        @pl.when(c >= P)
        def _():
            out_store(c, buf).wait()

        def grp(g, carry):
            lane = lax.iota(jnp.int32, L)
            ids = lane + g * L
            a0 = jnp.zeros((L,), jnp.float32)
            a1 = jnp.zeros((L,), jnp.float32)
            a2 = jnp.zeros((L,), jnp.float32)
            a3 = jnp.zeros((L,), jnp.float32)
            for t in range(DP):
                fv = (lane + t) & (DP - 1)
                sp = plsc.load_gather(srows, [ids, fv])
                dp = plsc.load_gather(drows, [ids, fv])
                p = plsc.bitcast(sp, jnp.bfloat16) * plsc.bitcast(dp, jnp.bfloat16)
                plo, phi = plsc.unpack(p, format=plsc.PackFormat.INTERLEAVED)
                if t % 2 == 0:
                    a0 = a0 + plo
                    a1 = a1 + phi
                else:
                    a2 = a2 + plo
                    a3 = a3 + phi
            outb[pl.ds(g * L, L)] = (a0 + a2) + (a1 + a3)
            return carry

        lax.fori_loop(0, C // L, grp, 0)
        out_store(c, buf).start()
        @pl.when(c + P < NCHUNK)
        def _():
            ia, ib = idx_copies(c + P, buf)
            ia.wait()
            ib.wait()
            ns, nd = gathers(buf)
            ns.start()
            nd.start()

    c0a, c0b = idx_copies(0, bufs[0])
    c1a, c1b = idx_copies(1, bufs[1])
    c0a.start()
    c0b.start()
    c1a.start()
    c1b.start()
    c0a.wait()
    c0b.wait()
    g0s, g0d = gathers(bufs[0])
    g0s.start()
    g0d.start()
    c1a.wait()
    c1b.wait()
    g1s, g1d = gathers(bufs[1])
    g1s.start()
    g1d.start()

    def pair(k, carry):
        process(2 * k, bufs[0])
        process(2 * k + 1, bufs[1])
        return carry

    lax.fori_loop(0, (NCHUNK - 1) // 2, pair, 0)
    process(NCHUNK - 1, bufs[0])
    out_store(0, bufs[0]).wait()
    out_store(0, bufs[1]).wait()


def kernel(h, edge_index):
    hp = jax.lax.bitcast_convert_type(
        h.astype(jnp.bfloat16).reshape(h.shape[0], DP, 2), jnp.int32)
    src = edge_index[0].astype(jnp.int32)
    dst = edge_index[1].astype(jnp.int32)
    return _edge_dot(hp, src, dst)


# speedup vs baseline: 10.2030x; 2.3492x over previous
import functools

import jax
import jax.numpy as jnp
from jax import lax
from jax.experimental import pallas as pl
from jax.experimental.pallas import tpu as pltpu
from jax.experimental.pallas import tpu_sc as plsc

NC, NS, L = 2, 16, 16
NW = NC * NS
E = 320000
D = 128
EPW = E // NW
C = 80
NCHUNK = EPW // C
P = 2
DP = D // 2


@functools.partial(
    pl.kernel,
    out_type=jax.ShapeDtypeStruct((E,), jnp.float32),
    mesh=plsc.VectorSubcoreMesh(
        core_axis_name="c", subcore_axis_name="s",
        num_cores=NC, num_subcores=NS),
    scratch_types=[
        pltpu.VMEM((2, C), jnp.int32),
        pltpu.VMEM((2, C), jnp.int32),
        pltpu.VMEM((C, DP), jnp.int32),
        pltpu.VMEM((C, DP), jnp.int32),
        pltpu.VMEM((C, DP), jnp.int32),
        pltpu.VMEM((C, DP), jnp.int32),
        pltpu.VMEM((C,), jnp.float32),
        pltpu.VMEM((C,), jnp.float32),
        pltpu.SemaphoreType.DMA,
        pltpu.SemaphoreType.DMA,
        pltpu.SemaphoreType.DMA,
        pltpu.SemaphoreType.DMA,
        pltpu.SemaphoreType.DMA,
        pltpu.SemaphoreType.DMA,
        pltpu.SemaphoreType.DMA,
        pltpu.SemaphoreType.DMA,
    ],
    compiler_params=pltpu.CompilerParams(
        needs_layout_passes=False, use_tc_tiling_on_sc=False),
)
def _edge_dot(h_hbm, src_hbm, dst_hbm, out_hbm,
              idx0, idx1, s0, d0, s1, d1, o0, o1,
              si0, ss0, sd0, so0, si1, ss1, sd1, so1):
    wid = lax.axis_index("s") * NC + lax.axis_index("c")
    wbase = wid * EPW
    bufs = ((idx0, s0, d0, o0, si0, ss0, sd0, so0),
            (idx1, s1, d1, o1, si1, ss1, sd1, so1))

    def idx_copies(c, buf):
        idx, _, _, _, si = buf[:5]
        base = wbase + c * C
        return (pltpu.make_async_copy(src_hbm.at[pl.ds(base, C)],
                                      idx.at[0], si),
                pltpu.make_async_copy(dst_hbm.at[pl.ds(base, C)],
                                      idx.at[1], si))

    def gathers(buf):
        idx, srows, drows, _, _, ss, sd = buf[:7]
        return (pltpu.make_async_copy(h_hbm.at[idx.at[0]], srows, ss),
                pltpu.make_async_copy(h_hbm.at[idx.at[1]], drows, sd))

    def out_store(c, buf):
        outb, so = buf[3], buf[7]
        return pltpu.make_async_copy(
            outb, out_hbm.at[pl.ds(wbase + c * C, C)], so)

    def process(c, buf):
        idx, srows, drows, outb = buf[:4]
        cs, cd = gathers(buf)
        cs.wait()
        cd.wait()
        @pl.when(c + P < NCHUNK)
        def _():
            ia, ib = idx_copies(c + P, buf)
            ia.start()
            ib.start()
        @pl.when(c >= P)
        def _():
            out_store(c, buf).wait()

        def grp2(base_edge, n_groups):
            lane = lax.iota(jnp.int32, L)
            idsg = [lane + base_edge + k * L for k in range(n_groups)]
            accs = [jnp.zeros((L,), jnp.float32)
                    for _ in range(4 * n_groups)]
            for t in range(DP):
                fv = (lane + t) & (DP - 1)
                for k in range(n_groups):
                    sp = plsc.load_gather(srows, [idsg[k], fv])
                    dp = plsc.load_gather(drows, [idsg[k], fv])
                    p = (plsc.bitcast(sp, jnp.bfloat16)
                         * plsc.bitcast(dp, jnp.bfloat16))
                    plo, phi = plsc.unpack(
                        p, format=plsc.PackFormat.INTERLEAVED)
                    j = 4 * k + 2 * (t % 2)
                    accs[j] = accs[j] + plo
                    accs[j + 1] = accs[j + 1] + phi
            for k in range(n_groups):
                a0, a1, a2, a3 = accs[4 * k:4 * k + 4]
                outb[pl.ds(base_edge + k * L, L)] = (a0 + a2) + (a1 + a3)

        def dgrp(j, carry):
            grp2(j * 2 * L, 2)
            return carry

        lax.fori_loop(0, C // (2 * L), dgrp, 0)
        grp2((C // (2 * L)) * 2 * L, 1)
        out_store(c, buf).start()
        @pl.when(c + P < NCHUNK)
        def _():
            ia, ib = idx_copies(c + P, buf)
            ia.wait()
            ib.wait()
            ns, nd = gathers(buf)
            ns.start()
            nd.start()

    c0a, c0b = idx_copies(0, bufs[0])
    c1a, c1b = idx_copies(1, bufs[1])
    c0a.start()
    c0b.start()
    c1a.start()
    c1b.start()
    c0a.wait()
    c0b.wait()
    g0s, g0d = gathers(bufs[0])
    g0s.start()
    g0d.start()
    c1a.wait()
    c1b.wait()
    g1s, g1d = gathers(bufs[1])
    g1s.start()
    g1d.start()

    def pair(k, carry):
        process(2 * k, bufs[0])
        process(2 * k + 1, bufs[1])
        return carry

    lax.fori_loop(0, (NCHUNK - 1) // 2, pair, 0)
    process(NCHUNK - 1, bufs[0])
    out_store(0, bufs[0]).wait()
    out_store(0, bufs[1]).wait()


def kernel(h, edge_index):
    hp = jax.lax.bitcast_convert_type(
        h.astype(jnp.bfloat16).reshape(h.shape[0], DP, 2), jnp.int32)
    src = edge_index[0].astype(jnp.int32)
    dst = edge_index[1].astype(jnp.int32)
    return _edge_dot(hp, src, dst)


# speedup vs baseline: 11.2867x; 1.1062x over previous
import functools

import jax
import jax.numpy as jnp
from jax import lax
from jax.experimental import pallas as pl
from jax.experimental.pallas import tpu as pltpu
from jax.experimental.pallas import tpu_sc as plsc

NC, NS, L = 2, 16, 16
NW = NC * NS
E = 320000
D = 128
C = 128
NCHUNK = E // C
P = 2
DP = D // 2


@functools.partial(
    pl.kernel,
    out_type=jax.ShapeDtypeStruct((E,), jnp.float32),
    mesh=plsc.VectorSubcoreMesh(
        core_axis_name="c", subcore_axis_name="s",
        num_cores=NC, num_subcores=NS),
    scratch_types=[
        pltpu.VMEM((2, C), jnp.int32),
        pltpu.VMEM((2, C), jnp.int32),
        pltpu.VMEM((C, DP), jnp.int32),
        pltpu.VMEM((C, DP), jnp.int32),
        pltpu.VMEM((C, DP), jnp.int32),
        pltpu.VMEM((C, DP), jnp.int32),
        pltpu.VMEM((C,), jnp.float32),
        pltpu.VMEM((C,), jnp.float32),
        pltpu.SemaphoreType.DMA,
        pltpu.SemaphoreType.DMA,
        pltpu.SemaphoreType.DMA,
        pltpu.SemaphoreType.DMA,
        pltpu.SemaphoreType.DMA,
        pltpu.SemaphoreType.DMA,
        pltpu.SemaphoreType.DMA,
        pltpu.SemaphoreType.DMA,
    ],
    compiler_params=pltpu.CompilerParams(
        needs_layout_passes=False, use_tc_tiling_on_sc=False),
)
def _edge_dot(h_hbm, src_hbm, dst_hbm, out_hbm,
              idx0, idx1, s0, d0, s1, d1, o0, o1,
              si0, ss0, sd0, so0, si1, ss1, sd1, so1):
    wid = lax.axis_index("s") * NC + lax.axis_index("c")
    start = lax.shift_right_logical(wid * NCHUNK, 5)
    end = lax.shift_right_logical((wid + 1) * NCHUNK, 5)
    bufs = ((idx0, s0, d0, o0, si0, ss0, sd0, so0),
            (idx1, s1, d1, o1, si1, ss1, sd1, so1))

    def idx_copies(c, buf):
        idx, _, _, _, si = buf[:5]
        base = c * C
        return (pltpu.make_async_copy(src_hbm.at[pl.ds(base, C)],
                                      idx.at[0], si),
                pltpu.make_async_copy(dst_hbm.at[pl.ds(base, C)],
                                      idx.at[1], si))

    def gathers(buf):
        idx, srows, drows, _, _, ss, sd = buf[:7]
        return (pltpu.make_async_copy(h_hbm.at[idx.at[0]], srows, ss),
                pltpu.make_async_copy(h_hbm.at[idx.at[1]], drows, sd))

    def out_store(c, buf):
        outb, so = buf[3], buf[7]
        return pltpu.make_async_copy(
            outb, out_hbm.at[pl.ds(c * C, C)], so)

    def process(c, buf):
        idx, srows, drows, outb = buf[:4]
        cs, cd = gathers(buf)
        cs.wait()
        cd.wait()
        @pl.when(c + P < end)
        def _():
            ia, ib = idx_copies(c + P, buf)
            ia.start()
            ib.start()
        @pl.when(c >= start + P)
        def _():
            out_store(c, buf).wait()

        def grp2(base_edge, n_groups):
            lane = lax.iota(jnp.int32, L)
            idsg = [lane + base_edge + k * L for k in range(n_groups)]
            accs = [jnp.zeros((L,), jnp.float32)
                    for _ in range(4 * n_groups)]
            for t in range(DP):
                fv = (lane + t) & (DP - 1)
                for k in range(n_groups):
                    sp = plsc.load_gather(srows, [idsg[k], fv])
                    dp = plsc.load_gather(drows, [idsg[k], fv])
                    p = (plsc.bitcast(sp, jnp.bfloat16)
                         * plsc.bitcast(dp, jnp.bfloat16))
                    plo, phi = plsc.unpack(
                        p, format=plsc.PackFormat.INTERLEAVED)
                    j = 4 * k + 2 * (t % 2)
                    accs[j] = accs[j] + plo
                    accs[j + 1] = accs[j + 1] + phi
            for k in range(n_groups):
                a0, a1, a2, a3 = accs[4 * k:4 * k + 4]
                outb[pl.ds(base_edge + k * L, L)] = (a0 + a2) + (a1 + a3)

        def dgrp(j, carry):
            grp2(j * 2 * L, 2)
            return carry

        lax.fori_loop(0, C // (2 * L), dgrp, 0)
        out_store(c, buf).start()
        @pl.when(c + P < end)
        def _():
            ia, ib = idx_copies(c + P, buf)
            ia.wait()
            ib.wait()
            ns, nd = gathers(buf)
            ns.start()
            nd.start()

    c0a, c0b = idx_copies(start, bufs[0])
    c1a, c1b = idx_copies(start + 1, bufs[1])
    c0a.start()
    c0b.start()
    c1a.start()
    c1b.start()
    c0a.wait()
    c0b.wait()
    g0s, g0d = gathers(bufs[0])
    g0s.start()
    g0d.start()
    c1a.wait()
    c1b.wait()
    g1s, g1d = gathers(bufs[1])
    g1s.start()
    g1d.start()

    def pair(k, carry):
        process(start + 2 * k, bufs[0])
        process(start + 2 * k + 1, bufs[1])
        return carry

    n = end - start
    lax.fori_loop(0, lax.shift_right_logical(n, 1), pair, 0)

    @pl.when(n & 1 == 1)
    def _():
        process(end - 1, bufs[0])

    out_store(start, bufs[0]).wait()
    out_store(start, bufs[1]).wait()


def kernel(h, edge_index):
    hp = jax.lax.bitcast_convert_type(
        h.astype(jnp.bfloat16).reshape(h.shape[0], DP, 2), jnp.int32)
    src = edge_index[0].astype(jnp.int32)
    dst = edge_index[1].astype(jnp.int32)
    return _edge_dot(hp, src, dst)


# speedup vs baseline: 12.4213x; 1.1005x over previous
import functools

import jax
import jax.numpy as jnp
from jax import lax
from jax.experimental import pallas as pl
from jax.experimental.pallas import tpu as pltpu
from jax.experimental.pallas import tpu_sc as plsc

NC, NS, L = 2, 16, 16
NW = NC * NS
E = 320000
D = 128
C = 128
NCHUNK = E // C
P = 3
DP = D // 2


@functools.partial(
    pl.kernel,
    out_type=jax.ShapeDtypeStruct((E,), jnp.float32),
    mesh=plsc.VectorSubcoreMesh(
        core_axis_name="c", subcore_axis_name="s",
        num_cores=NC, num_subcores=NS),
    scratch_types=[
        pltpu.VMEM((2, C), jnp.int32),
        pltpu.VMEM((2, C), jnp.int32),
        pltpu.VMEM((2, C), jnp.int32),
        pltpu.VMEM((C, DP), jnp.int32),
        pltpu.VMEM((C, DP), jnp.int32),
        pltpu.VMEM((C, DP), jnp.int32),
        pltpu.VMEM((C, DP), jnp.int32),
        pltpu.VMEM((C, DP), jnp.int32),
        pltpu.VMEM((C, DP), jnp.int32),
        pltpu.VMEM((C,), jnp.float32),
        pltpu.VMEM((C,), jnp.float32),
        pltpu.VMEM((C,), jnp.float32),
    ] + [pltpu.SemaphoreType.DMA] * 12,
    compiler_params=pltpu.CompilerParams(
        needs_layout_passes=False, use_tc_tiling_on_sc=False),
)
def _edge_dot(h_hbm, src_hbm, dst_hbm, out_hbm,
              idx0, idx1, idx2, s0, d0, s1, d1, s2, d2, o0, o1, o2,
              si0, ss0, sd0, so0, si1, ss1, sd1, so1, si2, ss2, sd2, so2):
    wid = lax.axis_index("s") * NC + lax.axis_index("c")
    start = lax.shift_right_logical(wid * NCHUNK, 5)
    end = lax.shift_right_logical((wid + 1) * NCHUNK, 5)
    bufs = ((idx0, s0, d0, o0, si0, ss0, sd0, so0),
            (idx1, s1, d1, o1, si1, ss1, sd1, so1),
            (idx2, s2, d2, o2, si2, ss2, sd2, so2))

    def idx_copies(c, buf):
        idx, _, _, _, si = buf[:5]
        base = c * C
        return (pltpu.make_async_copy(src_hbm.at[pl.ds(base, C)],
                                      idx.at[0], si),
                pltpu.make_async_copy(dst_hbm.at[pl.ds(base, C)],
                                      idx.at[1], si))

    def gathers(buf):
        idx, srows, drows, _, _, ss, sd = buf[:7]
        return (pltpu.make_async_copy(h_hbm.at[idx.at[0]], srows, ss),
                pltpu.make_async_copy(h_hbm.at[idx.at[1]], drows, sd))

    def out_store(c, buf):
        outb, so = buf[3], buf[7]
        return pltpu.make_async_copy(
            outb, out_hbm.at[pl.ds(c * C, C)], so)

    def process(c, buf):
        idx, srows, drows, outb = buf[:4]
        cs, cd = gathers(buf)
        cs.wait()
        cd.wait()
        @pl.when(c + P < end)
        def _():
            ia, ib = idx_copies(c + P, buf)
            ia.start()
            ib.start()
        @pl.when(c >= start + P)
        def _():
            out_store(c, buf).wait()

        def grp2(base_edge, n_groups):
            lane = lax.iota(jnp.int32, L)
            idsg = [lane + base_edge + k * L for k in range(n_groups)]
            accs = [jnp.zeros((L,), jnp.float32)
                    for _ in range(4 * n_groups)]
            for t in range(DP):
                if t <= DP - L:
                    fv = lane + t
                else:
                    fv = (lane + t) & (DP - 1)
                for k in range(n_groups):
                    sp = plsc.load_gather(srows, [idsg[k], fv])
                    dp = plsc.load_gather(drows, [idsg[k], fv])
                    p = (plsc.bitcast(sp, jnp.bfloat16)
                         * plsc.bitcast(dp, jnp.bfloat16))
                    plo, phi = plsc.unpack(
                        p, format=plsc.PackFormat.INTERLEAVED)
                    j = 4 * k + 2 * (t % 2)
                    accs[j] = accs[j] + plo
                    accs[j + 1] = accs[j + 1] + phi
            for k in range(n_groups):
                a0, a1, a2, a3 = accs[4 * k:4 * k + 4]
                outb[pl.ds(base_edge + k * L, L)] = (a0 + a2) + (a1 + a3)

        def dgrp(j, carry):
            grp2(j * 2 * L, 2)
            return carry

        lax.fori_loop(0, C // (2 * L), dgrp, 0)
        out_store(c, buf).start()
        @pl.when(c + P < end)
        def _():
            ia, ib = idx_copies(c + P, buf)
            ia.wait()
            ib.wait()
            ns, nd = gathers(buf)
            ns.start()
            nd.start()

    prim = [idx_copies(start + b, bufs[b]) for b in range(P)]
    for ca, cb in prim:
        ca.start()
        cb.start()
    for b, (ca, cb) in enumerate(prim):
        ca.wait()
        cb.wait()
        gs, gd = gathers(bufs[b])
        gs.start()
        gd.start()

    def triple(k, carry):
        process(start + P * k, bufs[0])
        process(start + P * k + 1, bufs[1])
        process(start + P * k + 2, bufs[2])
        return carry

    n = end - start
    ntrip = n // P
    lax.fori_loop(0, ntrip, triple, 0)

    rem = n - ntrip * P
    @pl.when(rem >= 1)
    def _():
        process(start + ntrip * P, bufs[0])
    @pl.when(rem >= 2)
    def _():
        process(start + ntrip * P + 1, bufs[1])

    for b in range(P):
        out_store(start, bufs[b]).wait()


def kernel(h, edge_index):
    hp = jax.lax.bitcast_convert_type(
        h.astype(jnp.bfloat16).reshape(h.shape[0], DP, 2), jnp.int32)
    src = edge_index[0].astype(jnp.int32)
    dst = edge_index[1].astype(jnp.int32)
    return _edge_dot(hp, src, dst)


# speedup vs baseline: 12.4768x; 1.0045x over previous
import functools

import jax
import jax.numpy as jnp
from jax import lax
from jax.experimental import pallas as pl
from jax.experimental.pallas import tpu as pltpu
from jax.experimental.pallas import tpu_sc as plsc

NC, NS, L = 2, 16, 16
NW = NC * NS
E = 320000
D = 128
C = 128
NCHUNK = E // C
P = 4
DP = D // 2


@functools.partial(
    pl.kernel,
    out_type=jax.ShapeDtypeStruct((E,), jnp.float32),
    mesh=plsc.VectorSubcoreMesh(
        core_axis_name="c", subcore_axis_name="s",
        num_cores=NC, num_subcores=NS),
    scratch_types=[
        pltpu.VMEM((2, C), jnp.int32),
        pltpu.VMEM((2, C), jnp.int32),
        pltpu.VMEM((2, C), jnp.int32),
        pltpu.VMEM((2, C), jnp.int32),
        pltpu.VMEM((C, DP), jnp.int32),
        pltpu.VMEM((C, DP), jnp.int32),
        pltpu.VMEM((C, DP), jnp.int32),
        pltpu.VMEM((C, DP), jnp.int32),
        pltpu.VMEM((C, DP), jnp.int32),
        pltpu.VMEM((C, DP), jnp.int32),
        pltpu.VMEM((C, DP), jnp.int32),
        pltpu.VMEM((C, DP), jnp.int32),
        pltpu.VMEM((C,), jnp.float32),
        pltpu.VMEM((C,), jnp.float32),
        pltpu.VMEM((C,), jnp.float32),
        pltpu.VMEM((C,), jnp.float32),
    ] + [pltpu.SemaphoreType.DMA] * 16,
    compiler_params=pltpu.CompilerParams(
        needs_layout_passes=False, use_tc_tiling_on_sc=False),
)
def _edge_dot(h_hbm, src_hbm, dst_hbm, out_hbm,
              idx0, idx1, idx2, idx3, s0, d0, s1, d1, s2, d2, s3, d3,
              o0, o1, o2, o3,
              si0, ss0, sd0, so0, si1, ss1, sd1, so1,
              si2, ss2, sd2, so2, si3, ss3, sd3, so3):
    wid = lax.axis_index("s") * NC + lax.axis_index("c")
    start = lax.shift_right_logical(wid * NCHUNK, 5)
    end = lax.shift_right_logical((wid + 1) * NCHUNK, 5)
    bufs = ((idx0, s0, d0, o0, si0, ss0, sd0, so0),
            (idx1, s1, d1, o1, si1, ss1, sd1, so1),
            (idx2, s2, d2, o2, si2, ss2, sd2, so2),
            (idx3, s3, d3, o3, si3, ss3, sd3, so3))

    def idx_copies(c, buf):
        idx, _, _, _, si = buf[:5]
        base = c * C
        return (pltpu.make_async_copy(src_hbm.at[pl.ds(base, C)],
                                      idx.at[0], si),
                pltpu.make_async_copy(dst_hbm.at[pl.ds(base, C)],
                                      idx.at[1], si))

    def gathers(buf):
        idx, srows, drows, _, _, ss, sd = buf[:7]
        return (pltpu.make_async_copy(h_hbm.at[idx.at[0]], srows, ss),
                pltpu.make_async_copy(h_hbm.at[idx.at[1]], drows, sd))

    def out_store(c, buf):
        outb, so = buf[3], buf[7]
        return pltpu.make_async_copy(
            outb, out_hbm.at[pl.ds(c * C, C)], so)

    def process(c, buf):
        idx, srows, drows, outb = buf[:4]
        cs, cd = gathers(buf)
        cs.wait()
        cd.wait()
        @pl.when(c + P < end)
        def _():
            ia, ib = idx_copies(c + P, buf)
            ia.start()
            ib.start()
        @pl.when(c >= start + P)
        def _():
            out_store(c, buf).wait()

        def grp2(base_edge, n_groups):
            lane = lax.iota(jnp.int32, L)
            idsg = [lane + base_edge + k * L for k in range(n_groups)]
            accs = [jnp.zeros((L,), jnp.float32)
                    for _ in range(4 * n_groups)]
            for t in range(DP):
                if t <= DP - L:
                    fv = lane + t
                else:
                    fv = (lane + t) & (DP - 1)
                for k in range(n_groups):
                    sp = plsc.load_gather(srows, [idsg[k], fv])
                    dp = plsc.load_gather(drows, [idsg[k], fv])
                    p = (plsc.bitcast(sp, jnp.bfloat16)
                         * plsc.bitcast(dp, jnp.bfloat16))
                    plo, phi = plsc.unpack(
                        p, format=plsc.PackFormat.INTERLEAVED)
                    j = 4 * k + 2 * (t % 2)
                    accs[j] = accs[j] + plo
                    accs[j + 1] = accs[j + 1] + phi
            for k in range(n_groups):
                a0, a1, a2, a3 = accs[4 * k:4 * k + 4]
                outb[pl.ds(base_edge + k * L, L)] = (a0 + a2) + (a1 + a3)

        def dgrp(j, carry):
            grp2(j * 2 * L, 2)
            return carry

        lax.fori_loop(0, C // (2 * L), dgrp, 0)
        out_store(c, buf).start()
        @pl.when(c + P < end)
        def _():
            ia, ib = idx_copies(c + P, buf)
            ia.wait()
            ib.wait()
            ns, nd = gathers(buf)
            ns.start()
            nd.start()

    prim = [idx_copies(start + b, bufs[b]) for b in range(P)]
    for ca, cb in prim:
        ca.start()
        cb.start()
    for b, (ca, cb) in enumerate(prim):
        ca.wait()
        cb.wait()
        gs, gd = gathers(bufs[b])
        gs.start()
        gd.start()

    def triple(k, carry):
        process(start + P * k, bufs[0])
        process(start + P * k + 1, bufs[1])
        process(start + P * k + 2, bufs[2])
        process(start + P * k + 3, bufs[3])
        return carry

    n = end - start
    ntrip = n // P
    lax.fori_loop(0, ntrip, triple, 0)

    rem = n - ntrip * P
    @pl.when(rem >= 1)
    def _():
        process(start + ntrip * P, bufs[0])
    @pl.when(rem >= 2)
    def _():
        process(start + ntrip * P + 1, bufs[1])
    @pl.when(rem >= 3)
    def _():
        process(start + ntrip * P + 2, bufs[2])

    for b in range(P):
        out_store(start, bufs[b]).wait()


def kernel(h, edge_index):
    hp = jax.lax.bitcast_convert_type(
        h.astype(jnp.bfloat16).reshape(h.shape[0], DP, 2), jnp.int32)
    src = edge_index[0].astype(jnp.int32)
    dst = edge_index[1].astype(jnp.int32)
    return _edge_dot(hp, src, dst)
